# baseline (speedup 1.0000x reference)
import jax
import jax.numpy as jnp
from jax.experimental import pallas as pl
from jax.experimental.pallas import tpu as pltpu

_SPB = 4
_TT = 512


def _body(enc_ref, dec_ref, Wpre_ref, bpre_ref, cbT_ref, cb_ref, Wpost_ref,
          bpost_ref, out_ref, cm_ref, kl_ref, cbW_ref, cbT2_ref, c2_ref,
          Wpost_bf_ref, counts_ref):
    TT, D = enc_ref.shape[1], enc_ref.shape[2]
    K = cb_ref.shape[0]
    b = pl.program_id(0)
    t = pl.program_id(1)
    nt = pl.num_programs(1)
    T = TT * nt

    @pl.when(jnp.logical_and(b == 0, t == 0))
    def _init():
        cb = cb_ref[...]
        Wpost = Wpost_ref[...]
        cbW_ref[...] = jnp.dot(cb, Wpost,
                               preferred_element_type=jnp.float32
                               ).astype(jnp.bfloat16)
        Wpost_bf_ref[...] = Wpost.astype(jnp.bfloat16)
        cbT = cbT_ref[...]
        cbT2_ref[...] = cbT * -2.0
        c2_ref[...] = jnp.sum(cbT * cbT, axis=0, keepdims=True)

    iota = jax.lax.broadcasted_iota(
        jnp.int32, (TT, K), 1).astype(jnp.float32)
    ones_row = jnp.ones((8, TT), dtype=jnp.bfloat16)
    for i in range(_SPB):
        e = enc_ref[i]
        d = dec_ref[i]
        r = jnp.dot(e - d, Wpre_ref[...],
                    preferred_element_type=jnp.float32) + bpre_ref[...]
        scores = jnp.dot(r, cbT2_ref[...],
                         preferred_element_type=jnp.float32) + c2_ref[...]
        m = jnp.min(scores, axis=1)
        codes = jnp.min(jnp.where(scores <= m[:, None], iota, float(K)),
                        axis=1)
        onehot = jnp.where(iota == codes[:, None], 1.0, 0.0
                           ).astype(jnp.bfloat16)
        qW = jnp.dot(onehot, cbW_ref[...],
                     preferred_element_type=jnp.float32)

        cm_part = (jnp.sum(r * r) + jnp.sum(m)) / (T * D)
        counts = jnp.dot(ones_row, onehot,
                         preferred_element_type=jnp.float32)[0]

        @pl.when(t == 0)
        def _first():
            counts_ref[i, 0, :] = counts
            cm_ref[i, 0, :] = jnp.full((128,), cm_part, dtype=jnp.float32)

        @pl.when(t != 0)
        def _rest():
            counts_ref[i, 0, :] = counts_ref[i, 0, :] + counts
            cm_ref[i, 0, :] = cm_ref[i, 0, :] + cm_part

        @pl.when(t == nt - 1)
        def _last():
            p = counts_ref[i, 0, :] * (1.0 / T)
            klv = jnp.sum(p * jnp.log(p * K + 1e-10))
            kl_ref[i, 0, :] = jnp.full((128,), klv, dtype=jnp.float32)

        out_ref[i] = (jnp.dot(d.astype(jnp.bfloat16), Wpost_bf_ref[...],
                              preferred_element_type=jnp.float32)
                      + qW + bpost_ref[...])


def kernel(enc, dec, W_pre, b_pre, codebook, W_post, b_post):
    B, T, D = enc.shape
    K = codebook.shape[0]
    cbT = codebook.T
    bpre2 = b_pre.reshape(1, D)
    bpost2 = b_post.reshape(1, D)

    out_shapes = (
        jax.ShapeDtypeStruct((B, T, D), jnp.float32),
        jax.ShapeDtypeStruct((B, 1, 128), jnp.float32),
        jax.ShapeDtypeStruct((B, 1, 128), jnp.float32),
    )
    full = lambda shape: pl.BlockSpec(shape, lambda b, t: (0,) * len(shape))
    dec_refine, cm2, kl2 = pl.pallas_call(
        _body,
        grid=(B // _SPB, T // _TT),
        in_specs=[
            pl.BlockSpec((_SPB, _TT, D), lambda b, t: (b, t, 0)),
            pl.BlockSpec((_SPB, _TT, D), lambda b, t: (b, t, 0)),
            full((D, D)),
            full((1, D)),
            full((D, K)),
            full((K, D)),
            full((D, D)),
            full((1, D)),
        ],
        out_specs=(
            pl.BlockSpec((_SPB, _TT, D), lambda b, t: (b, t, 0)),
            pl.BlockSpec((_SPB, 1, 128), lambda b, t: (b, 0, 0)),
            pl.BlockSpec((_SPB, 1, 128), lambda b, t: (b, 0, 0)),
        ),
        scratch_shapes=[
            pltpu.VMEM((K, D), jnp.bfloat16),
            pltpu.VMEM((D, K), jnp.float32),
            pltpu.VMEM((1, K), jnp.float32),
            pltpu.VMEM((D, D), jnp.bfloat16),
            pltpu.VMEM((_SPB, 1, K), jnp.float32),
        ],
        out_shape=out_shapes,
    )(enc, dec, W_pre, bpre2, cbT, codebook, W_post, bpost2)

    cm = cm2[:, 0, 0]
    kl = kl2[:, 0, 0]
    return dec_refine, cm, cm, kl


# speedup vs baseline: 1.2684x; 1.2684x over previous
import jax
import jax.numpy as jnp
from jax.experimental import pallas as pl
from jax.experimental.pallas import tpu as pltpu

_SPB = 4
_KS = 2


def _body(enc_ref, dec_ref, Wpre_ref, bpre_ref, cbT_ref, cb_ref, Wpost_ref,
          bpost_ref, out_ref, cm_ref, kl_ref, cbW_ref, cbT2_ref, c2_ref,
          Wpost_bf_ref):
    T, D = enc_ref.shape[1], enc_ref.shape[2]
    K = cb_ref.shape[0]
    KH = K // _KS
    b = pl.program_id(0)

    @pl.when(b == 0)
    def _init():
        cb = cb_ref[...]
        Wpost = Wpost_ref[...]
        cbW_ref[...] = jnp.dot(cb, Wpost,
                               preferred_element_type=jnp.float32
                               ).astype(jnp.bfloat16)
        Wpost_bf_ref[...] = Wpost.astype(jnp.bfloat16)
        cbT = cbT_ref[...]
        cbT2_ref[...] = cbT * -2.0
        c2_ref[...] = jnp.sum(cbT * cbT, axis=0, keepdims=True)

    iota = jax.lax.broadcasted_iota(
        jnp.int32, (T, KH), 1).astype(jnp.float32)
    ones_row = jnp.ones((8, T), dtype=jnp.bfloat16)
    for i in range(_SPB):
        e = enc_ref[i]
        d = dec_ref[i]
        r = jnp.dot(e - d, Wpre_ref[...],
                    preferred_element_type=jnp.float32) + bpre_ref[...]
        ss = []
        for j in range(_KS):
            s = jnp.dot(r, cbT2_ref[:, j * KH:(j + 1) * KH],
                        preferred_element_type=jnp.float32
                        ) + c2_ref[:, j * KH:(j + 1) * KH]
            ss.append(s)
        m = jnp.min(ss[0], axis=1)
        for j in range(1, _KS):
            m = jnp.minimum(m, jnp.min(ss[j], axis=1))
        code_parts = [
            jnp.min(jnp.where(ss[j] <= m[:, None], iota + float(j * KH),
                              float(K)), axis=1)
            for j in range(_KS)
        ]
        codes = code_parts[0]
        for j in range(1, _KS):
            codes = jnp.minimum(codes, code_parts[j])

        qW = jnp.zeros((T, D), dtype=jnp.float32)
        klv = jnp.float32(0.0)
        msum = jnp.sum(m)
        for j in range(_KS):
            onehot = jnp.where(iota + float(j * KH) == codes[:, None],
                               1.0, 0.0).astype(jnp.bfloat16)
            qW = qW + jnp.dot(onehot, cbW_ref[j * KH:(j + 1) * KH, :],
                              preferred_element_type=jnp.float32)
            counts = jnp.dot(ones_row, onehot,
                             preferred_element_type=jnp.float32)[0]
            p = counts * (1.0 / T)
            klv = klv + jnp.sum(p * jnp.log(p * K + 1e-10))

        cm = (jnp.sum(r * r) + msum) / (T * D)
        cm_ref[i, 0, :] = jnp.full((128,), cm, dtype=jnp.float32)
        kl_ref[i, 0, :] = jnp.full((128,), klv, dtype=jnp.float32)
        out_ref[i] = (jnp.dot(d.astype(jnp.bfloat16), Wpost_bf_ref[...],
                              preferred_element_type=jnp.float32)
                      + qW + bpost_ref[...])


def kernel(enc, dec, W_pre, b_pre, codebook, W_post, b_post):
    B, T, D = enc.shape
    K = codebook.shape[0]
    cbT = codebook.T
    bpre2 = b_pre.reshape(1, D)
    bpost2 = b_post.reshape(1, D)

    out_shapes = (
        jax.ShapeDtypeStruct((B, T, D), jnp.float32),
        jax.ShapeDtypeStruct((B, 1, 128), jnp.float32),
        jax.ShapeDtypeStruct((B, 1, 128), jnp.float32),
    )
    full = lambda shape: pl.BlockSpec(shape, lambda b: (0,) * len(shape))
    dec_refine, cm2, kl2 = pl.pallas_call(
        _body,
        grid=(B // _SPB,),
        in_specs=[
            pl.BlockSpec((_SPB, T, D), lambda b: (b, 0, 0)),
            pl.BlockSpec((_SPB, T, D), lambda b: (b, 0, 0)),
            full((D, D)),
            full((1, D)),
            full((D, K)),
            full((K, D)),
            full((D, D)),
            full((1, D)),
        ],
        out_specs=(
            pl.BlockSpec((_SPB, T, D), lambda b: (b, 0, 0)),
            pl.BlockSpec((_SPB, 1, 128), lambda b: (b, 0, 0)),
            pl.BlockSpec((_SPB, 1, 128), lambda b: (b, 0, 0)),
        ),
        scratch_shapes=[
            pltpu.VMEM((K, D), jnp.bfloat16),
            pltpu.VMEM((D, K), jnp.float32),
            pltpu.VMEM((1, K), jnp.float32),
            pltpu.VMEM((D, D), jnp.bfloat16),
        ],
        out_shape=out_shapes,
    )(enc, dec, W_pre, bpre2, cbT, codebook, W_post, bpost2)

    cm = cm2[:, 0, 0]
    kl = kl2[:, 0, 0]
    return dec_refine, cm, cm, kl
